# baseline (speedup 1.0000x reference)
import functools

import jax
import jax.numpy as jnp
from jax import lax
from jax.experimental import pallas as pl
from jax.experimental.pallas import tpu as pltpu
from jax.experimental.pallas import tpu_sc as plsc

_LOG2 = 0.6931471805599453
_CUTOFF = 5.0

_NC = 2
_NS = 16
_NW = _NC * _NS


def _ssp(v):
    return jax.nn.softplus(v) - _LOG2


def _in2f_body(x_ref, w_ref, y_ref):
    y_ref[...] = jnp.dot(x_ref[...], w_ref[...],
                         preferred_element_type=jnp.float32)


def _in2f(x2d, Win):
    M, D = x2d.shape
    F = Win.shape[1]
    T = 1000
    return pl.pallas_call(
        _in2f_body,
        grid=(M // T,),
        in_specs=[
            pl.BlockSpec((T, D), lambda i: (i, 0)),
            pl.BlockSpec((D, F), lambda i: (0, 0)),
        ],
        out_specs=pl.BlockSpec((T, F), lambda i: (i, 0)),
        out_shape=jax.ShapeDtypeStruct((M, F), jnp.float32),
    )(x2d, Win)


def _sc_gather(table, idx3d):
    NW, NCH, CH = idx3d.shape
    F = table.shape[1]
    mesh = plsc.VectorSubcoreMesh(core_axis_name="c", subcore_axis_name="s",
                                  num_cores=_NC, num_subcores=_NS)

    @functools.partial(
        pl.kernel,
        out_type=jax.ShapeDtypeStruct((NW, NCH * CH, F), jnp.float32),
        mesh=mesh,
        scratch_types=[
            pltpu.VMEM((2, CH), jnp.int32),
            pltpu.VMEM((2, CH, F), jnp.float32),
            pltpu.SemaphoreType.DMA,
            pltpu.SemaphoreType.DMA,
        ],
    )
    def k(table_hbm, idx_hbm, out_hbm, idx_v, rows_v, gsem, ssem):
        w = lax.axis_index("s") * _NC + lax.axis_index("c")

        pltpu.sync_copy(idx_hbm.at[w, 0], idx_v.at[0])
        pltpu.async_copy(table_hbm.at[idx_v.at[0]], rows_v.at[0], gsem)

        def body(j, _):
            s = lax.rem(j, 2)
            o = lax.rem(j + 1, 2)

            @pl.when(j + 1 < NCH)
            def _():
                @pl.when(j >= 1)
                def _():
                    pltpu.make_async_copy(rows_v.at[o], out_hbm.at[w, pl.ds((j - 1) * CH, CH)],
                                          ssem).wait()
                pltpu.sync_copy(idx_hbm.at[w, j + 1], idx_v.at[o])
                pltpu.async_copy(table_hbm.at[idx_v.at[o]], rows_v.at[o],
                                 gsem)

            pltpu.make_async_copy(table_hbm.at[idx_v.at[s]], rows_v.at[s],
                                  gsem).wait()
            pltpu.async_copy(rows_v.at[s], out_hbm.at[w, pl.ds(j * CH, CH)], ssem)
            return 0

        lax.fori_loop(0, NCH, body, 0, unroll=False)
        if NCH >= 2:
            pltpu.make_async_copy(rows_v.at[lax.rem(NCH - 2, 2)],
                                  out_hbm.at[w, pl.ds((NCH - 2) * CH, CH)], ssem).wait()
        pltpu.make_async_copy(rows_v.at[lax.rem(NCH - 1, 2)],
                              out_hbm.at[w, pl.ds((NCH - 1) * CH, CH)], ssem).wait()

    return k(table, idx3d)


def _fused_body(f_ref, r_ref, m_ref, g_ref, w1_ref, b1_ref, w2_ref, b2_ref,
                wf_ref, bf_ref, wd_ref, bd_ref, o_ref, *, TN, NBR, S, F):
    f = f_ref[...].reshape(S, NBR * TN)
    h = _ssp(lax.dot_general(f, w1_ref[...], (((0,), (0,)), ((), ())),
                             preferred_element_type=jnp.float32)
             + b1_ref[...])
    w = jnp.dot(h, w2_ref[...], preferred_element_type=jnp.float32) \
        + b2_ref[...]
    r = r_ref[...].reshape(NBR, TN)
    c = 0.5 * (jnp.cos(r * (jnp.pi / _CUTOFF)) + 1.0)
    c = c * (r < _CUTOFF).astype(jnp.float32) * m_ref[...].reshape(NBR, TN)
    w3 = w.reshape(NBR, TN, F) * c[..., None]
    agg = jnp.sum(w3 * g_ref[...].reshape(NBR, TN, F), axis=0)
    a = _ssp(jnp.dot(agg, wf_ref[...], preferred_element_type=jnp.float32)
             + bf_ref[...])
    o_ref[...] = (jnp.dot(a, wd_ref[...], preferred_element_type=jnp.float32)
                  + bd_ref[...]).reshape(1, TN, F)


def _fused(fT, rT, mT, g4, W1, b1, W2, b2, Wf, bf, Wd, bd):
    B, S, NBR, N = fT.shape
    F = W2.shape[1]
    A = Wd.shape[1]
    TN = 128
    NT = pl.cdiv(N, TN)
    body = functools.partial(_fused_body, TN=TN, NBR=NBR, S=S, F=F)
    full = lambda b, t: (0, 0)
    return pl.pallas_call(
        body,
        grid=(B, NT),
        in_specs=[
            pl.BlockSpec((1, S, NBR, TN), lambda b, t: (b, 0, 0, t)),
            pl.BlockSpec((1, NBR, TN), lambda b, t: (b, 0, t)),
            pl.BlockSpec((1, NBR, TN), lambda b, t: (b, 0, t)),
            pl.BlockSpec((NBR, 1, TN, F), lambda b, t: (0, b, t, 0)),
            pl.BlockSpec((S, F), full),
            pl.BlockSpec((1, F), full),
            pl.BlockSpec((F, F), full),
            pl.BlockSpec((1, F), full),
            pl.BlockSpec((F, A), full),
            pl.BlockSpec((1, A), full),
            pl.BlockSpec((A, A), full),
            pl.BlockSpec((1, A), full),
        ],
        out_specs=pl.BlockSpec((1, TN, A), lambda b, t: (b, t, 0)),
        out_shape=jax.ShapeDtypeStruct((B, N, A), jnp.float32),
    )(fT, rT, mT, g4, W1, b1.reshape(1, -1), W2, b2.reshape(1, -1),
      Wf, bf.reshape(1, -1), Wd, bd.reshape(1, -1))


def kernel(x, r_ij, neighbors, neighbor_mask, f_ij, W1, b1, W2, b2, Win,
           Wf, bf, Wd, bd):
    B, N, NBR = neighbors.shape
    D = x.shape[-1]
    S = f_ij.shape[-1]
    M = B * N

    y = _in2f(x.reshape(M, D), Win)

    idxT = (neighbors.transpose(2, 0, 1)
            + (jnp.arange(B, dtype=jnp.int32) * N)[None, :, None])

    BH = B // 2
    per_w = BH * N * NBR // _NW
    CH = 40
    fT = f_ij.transpose(0, 3, 2, 1)
    rT = r_ij.transpose(0, 2, 1)
    mT = neighbor_mask.transpose(0, 2, 1)

    outs = []
    for hb in range(2):
        sl = slice(hb * BH, (hb + 1) * BH)
        idx3d = idxT[:, sl].reshape(_NW, per_w // CH, CH)
        g4 = _sc_gather(y, idx3d).reshape(NBR, BH, N, -1)
        outs.append((g4, sl))

    return jnp.concatenate(
        [_fused(fT[sl], rT[sl], mT[sl], g4, W1, b1, W2, b2, Wf, bf, Wd, bd)
         for g4, sl in outs], axis=0)


# speedup vs baseline: 1.1379x; 1.1379x over previous
import functools

import jax
import jax.numpy as jnp
from jax import lax
from jax.experimental import pallas as pl
from jax.experimental.pallas import tpu as pltpu
from jax.experimental.pallas import tpu_sc as plsc

_LOG2 = 0.6931471805599453
_CUTOFF = 5.0

_NC = 2
_NS = 16
_NW = _NC * _NS


def _ssp(v):
    av = jnp.abs(v)
    return jnp.maximum(v, 0.0) + (jnp.log(1.0 + jnp.exp(-av)) - _LOG2)


def _in2f_body(x_ref, w_ref, y_ref):
    y_ref[...] = jnp.dot(x_ref[...], w_ref[...],
                         preferred_element_type=jnp.float32)


def _in2f(x2d, Win):
    M, D = x2d.shape
    F = Win.shape[1]
    T = 1000
    return pl.pallas_call(
        _in2f_body,
        grid=(M // T,),
        in_specs=[
            pl.BlockSpec((T, D), lambda i: (i, 0)),
            pl.BlockSpec((D, F), lambda i: (0, 0)),
        ],
        out_specs=pl.BlockSpec((T, F), lambda i: (i, 0)),
        out_shape=jax.ShapeDtypeStruct((M, F), jnp.float32),
    )(x2d, Win)


def _sc_gather(table, idx3d):
    NW, NCH, CH = idx3d.shape
    F = table.shape[1]
    mesh = plsc.VectorSubcoreMesh(core_axis_name="c", subcore_axis_name="s",
                                  num_cores=_NC, num_subcores=_NS)

    @functools.partial(
        pl.kernel,
        out_type=jax.ShapeDtypeStruct((NW, NCH * CH, F), table.dtype),
        mesh=mesh,
        scratch_types=[
            pltpu.VMEM((2, CH), jnp.int32),
            pltpu.VMEM((2, CH, F), table.dtype),
            pltpu.SemaphoreType.DMA,
            pltpu.SemaphoreType.DMA,
        ],
    )
    def k(table_hbm, idx_hbm, out_hbm, idx_v, rows_v, gsem, ssem):
        w = lax.axis_index("s") * _NC + lax.axis_index("c")

        pltpu.sync_copy(idx_hbm.at[w, 0], idx_v.at[0])
        pltpu.async_copy(table_hbm.at[idx_v.at[0]], rows_v.at[0], gsem)

        def body(j, _):
            s = lax.rem(j, 2)
            o = lax.rem(j + 1, 2)

            @pl.when(j + 1 < NCH)
            def _():
                @pl.when(j >= 1)
                def _():
                    pltpu.make_async_copy(rows_v.at[o], out_hbm.at[w, pl.ds((j - 1) * CH, CH)],
                                          ssem).wait()
                pltpu.sync_copy(idx_hbm.at[w, j + 1], idx_v.at[o])
                pltpu.async_copy(table_hbm.at[idx_v.at[o]], rows_v.at[o],
                                 gsem)

            pltpu.make_async_copy(table_hbm.at[idx_v.at[s]], rows_v.at[s],
                                  gsem).wait()
            pltpu.async_copy(rows_v.at[s], out_hbm.at[w, pl.ds(j * CH, CH)], ssem)
            return 0

        lax.fori_loop(0, NCH, body, 0, unroll=False)
        if NCH >= 2:
            pltpu.make_async_copy(rows_v.at[lax.rem(NCH - 2, 2)],
                                  out_hbm.at[w, pl.ds((NCH - 2) * CH, CH)], ssem).wait()
        pltpu.make_async_copy(rows_v.at[lax.rem(NCH - 1, 2)],
                              out_hbm.at[w, pl.ds((NCH - 1) * CH, CH)], ssem).wait()

    return k(table, idx3d)


def _fused_body(f_ref, r_ref, m_ref, g_ref, w1_ref, b1_ref, w2_ref, b2_ref,
                wf_ref, bf_ref, wd_ref, bd_ref, o_ref, *, TN, NBR, S, F):
    f = f_ref[...].reshape(S, NBR * TN)
    h = _ssp(lax.dot_general(f, w1_ref[...], (((0,), (0,)), ((), ())),
                             preferred_element_type=jnp.float32)
             + b1_ref[...])
    w = jnp.dot(h, w2_ref[...], preferred_element_type=jnp.float32) \
        + b2_ref[...]
    r = r_ref[...].reshape(NBR, TN)
    c = 0.5 * (jnp.cos(r * (jnp.pi / _CUTOFF)) + 1.0)
    c = c * (r < _CUTOFF).astype(jnp.float32) * m_ref[...].reshape(NBR, TN)
    w3 = w.reshape(NBR, TN, F) * c[..., None]
    agg = jnp.sum(w3 * g_ref[...].reshape(NBR, TN, F), axis=0)
    a = _ssp(jnp.dot(agg, wf_ref[...], preferred_element_type=jnp.float32)
             + bf_ref[...])
    o_ref[...] = (jnp.dot(a, wd_ref[...], preferred_element_type=jnp.float32)
                  + bd_ref[...]).reshape(1, TN, F)


def _fused(fT, rT, mT, g4, W1, b1, W2, b2, Wf, bf, Wd, bd):
    B, S, NBR, N = fT.shape
    F = W2.shape[1]
    A = Wd.shape[1]
    TN = 128
    NT = pl.cdiv(N, TN)
    body = functools.partial(_fused_body, TN=TN, NBR=NBR, S=S, F=F)
    full = lambda b, t: (0, 0)
    return pl.pallas_call(
        body,
        grid=(B, NT),
        in_specs=[
            pl.BlockSpec((1, S, NBR, TN), lambda b, t: (b, 0, 0, t)),
            pl.BlockSpec((1, NBR, TN), lambda b, t: (b, 0, t)),
            pl.BlockSpec((1, NBR, TN), lambda b, t: (b, 0, t)),
            pl.BlockSpec((NBR, 1, TN, F), lambda b, t: (0, b, t, 0)),
            pl.BlockSpec((S, F), full),
            pl.BlockSpec((1, F), full),
            pl.BlockSpec((F, F), full),
            pl.BlockSpec((1, F), full),
            pl.BlockSpec((F, A), full),
            pl.BlockSpec((1, A), full),
            pl.BlockSpec((A, A), full),
            pl.BlockSpec((1, A), full),
        ],
        out_specs=pl.BlockSpec((1, TN, A), lambda b, t: (b, t, 0)),
        out_shape=jax.ShapeDtypeStruct((B, N, A), jnp.float32),
    )(fT, rT, mT, g4, W1, b1.reshape(1, -1), W2, b2.reshape(1, -1),
      Wf, bf.reshape(1, -1), Wd, bd.reshape(1, -1))


def kernel(x, r_ij, neighbors, neighbor_mask, f_ij, W1, b1, W2, b2, Win,
           Wf, bf, Wd, bd):
    B, N, NBR = neighbors.shape
    D = x.shape[-1]
    S = f_ij.shape[-1]
    M = B * N

    y = _in2f(x.reshape(M, D), Win)

    idxT = (neighbors.transpose(2, 0, 1)
            + (jnp.arange(B, dtype=jnp.int32) * N)[None, :, None])

    per_w = B * N * NBR // _NW
    CH = 80
    idx3d = idxT.reshape(_NW, per_w // CH, CH)

    g4 = _sc_gather(y, idx3d).reshape(NBR, B, N, -1)

    return _fused(f_ij.transpose(0, 3, 2, 1), r_ij.transpose(0, 2, 1),
                  neighbor_mask.transpose(0, 2, 1), g4,
                  W1, b1, W2, b2, Wf, bf, Wd, bd)
